# Initial kernel scaffold; baseline (speedup 1.0000x reference)
import jax
import jax.numpy as jnp
from jax.experimental import pallas as pl


def kernel(x, expert_indices, expert_weights, W1, b1, W2, b2):
    raise NotImplementedError("write your pallas kernel here")


# speedup vs baseline: 2.3484x; 2.3484x over previous
import functools

import jax
import jax.numpy as jnp
from jax import lax
from jax.experimental import pallas as pl
from jax.experimental.pallas import tpu as pltpu

B, T, D, E, K = 1, 2048, 1024, 8, 2
N = T * K
BLK = 256
NT_MAX = N // BLK + E
NPAD = NT_MAX * BLK
NF = 4
F = 4 * D
FBLK = F // NF


def _prep(expert_indices):
    e_flat = expert_indices.reshape(N).astype(jnp.int32)
    onehot = e_flat[:, None] == jnp.arange(E, dtype=jnp.int32)[None, :]
    counts = jnp.sum(onehot, axis=0, dtype=jnp.int32)
    rank = jnp.cumsum(onehot.astype(jnp.int32), axis=0)
    rank_i = jnp.take_along_axis(rank, e_flat[:, None], axis=1)[:, 0] - 1
    tiles_per_e = (counts + BLK - 1) // BLK
    tile_start_e = jnp.concatenate(
        [jnp.zeros(1, jnp.int32), jnp.cumsum(tiles_per_e, dtype=jnp.int32)])[:E]
    P = tile_start_e[e_flat] * BLK + rank_i
    tok = jnp.arange(N, dtype=jnp.int32) // K
    tok_padded = jnp.zeros(NPAD, jnp.int32).at[P].set(tok)
    pairid_padded = jnp.zeros(NPAD, jnp.int32).at[P].set(
        jnp.arange(N, dtype=jnp.int32))
    total_tiles = jnp.sum(tiles_per_e)
    tile_ids = jnp.arange(NT_MAX, dtype=jnp.int32)
    tile_expert = jnp.sum(tile_ids[:, None] >= tile_start_e[None, :],
                          axis=1, dtype=jnp.int32) - 1
    tile_expert = jnp.where(tile_ids < total_tiles, tile_expert, -1)
    p0 = P.reshape(T, K)[:, 0]
    p1 = P.reshape(T, K)[:, 1]
    return tok_padded, pairid_padded, tile_expert, p0, p1


def _ffn_body(te_ref, xs_ref, w1_ref, b1_ref, w2_ref, b2_ref, ws_ref,
              out_ref, acc_ref):
    fi = pl.program_id(1)
    j = pl.program_id(0)
    active = te_ref[j] >= 0

    @pl.when(active)
    def _():
        h = jnp.dot(xs_ref[...], w1_ref[0], preferred_element_type=jnp.float32)
        h = h + b1_ref[0, 0, 0][None, :]
        h = h * 0.5 * (1.0 + lax.erf(h * 0.7071067811865476))
        y = jnp.dot(h, w2_ref[0], preferred_element_type=jnp.float32)

        @pl.when(fi == 0)
        def _():
            acc_ref[...] = y

        @pl.when(fi > 0)
        def _():
            acc_ref[...] += y

        @pl.when(fi == NF - 1)
        def _():
            out_ref[...] = ((acc_ref[...] + b2_ref[0, 0][None, :])
                            * ws_ref[0, 0][:, None])


def _grouped_ffn(tile_expert, xs, W1, b1, W2, b2, wsorted):
    b1r = b1.reshape(E, NF, 1, FBLK)
    b2r = b2.reshape(E, 1, D)
    wsr = wsorted.reshape(NT_MAX, 1, BLK)
    grid_spec = pltpu.PrefetchScalarGridSpec(
        num_scalar_prefetch=1,
        grid=(NT_MAX, NF),
        in_specs=[
            pl.BlockSpec((BLK, D), lambda j, fi, te: (j, 0)),
            pl.BlockSpec((1, D, FBLK), lambda j, fi, te: (jnp.maximum(te[j], 0), 0, fi)),
            pl.BlockSpec((1, 1, 1, FBLK), lambda j, fi, te: (jnp.maximum(te[j], 0), fi, 0, 0)),
            pl.BlockSpec((1, FBLK, D), lambda j, fi, te: (jnp.maximum(te[j], 0), fi, 0)),
            pl.BlockSpec((1, 1, D), lambda j, fi, te: (jnp.maximum(te[j], 0), 0, 0)),
            pl.BlockSpec((1, 1, BLK), lambda j, fi, te: (j, 0, 0)),
        ],
        out_specs=pl.BlockSpec((BLK, D), lambda j, fi, te: (j, 0)),
        scratch_shapes=[pltpu.VMEM((BLK, D), jnp.float32)],
    )
    return pl.pallas_call(
        _ffn_body,
        grid_spec=grid_spec,
        out_shape=jax.ShapeDtypeStruct((NPAD, D), jnp.float32),
    )(tile_expert, xs, W1, b1r, W2, b2r, wsr)


def kernel(x, expert_indices, expert_weights, W1, b1, W2, b2):
    x2d = x.reshape(T, D)
    tok_padded, pairid_padded, tile_expert, p0, p1 = _prep(expert_indices)

    xs = x2d[tok_padded]
    wf = expert_weights.reshape(N)
    wa = wf[pairid_padded]
    wb = wf[pairid_padded ^ 1]
    m = jnp.maximum(wa, wb)
    ea = jnp.exp(wa - m)
    eb = jnp.exp(wb - m)
    wsorted = ea / (ea + eb)

    ysw = _grouped_ffn(tile_expert, xs, W1, b1, W2, b2, wsorted)

    out = x2d + ysw[p0] + ysw[p1]
    return out.reshape(B, T, D)
